# baseline (speedup 1.0000x reference)
import functools

import jax
import jax.numpy as jnp
from jax import lax
from jax.experimental import pallas as pl
from jax.experimental.pallas import tpu as pltpu
from jax.experimental.pallas import tpu_sc as plsc

N = 10000
E = 320000
F = 128
NEG_SLOPE = 0.2

NC = 2
NS = 16
NW = NC * NS
LANES = 16

NPAD = 10240
EPAD = 327680
PADNODE = 10200
C = 128
EPW = EPAD // NW
NCHUNK = EPW // C
G = 8
NG = NCHUNK // G
C1 = 1024
CH0 = 56
CH1 = 160 - CH0
STRIPE = NPAD // NS
RB = 1280


def _pre_body(h_ref, w_ref, asr_ref, adr_ref, xw_ref, as_ref, ad_ref):
    xw = jnp.dot(h_ref[...], w_ref[...], preferred_element_type=jnp.float32)
    xw_ref[...] = xw
    as_ref[...] = jnp.sum(xw * asr_ref[...], axis=1, keepdims=True)
    ad_ref[...] = jnp.sum(xw * adr_ref[...], axis=1, keepdims=True)


def _pre(h, W, asrc, adst):
    return pl.pallas_call(
        _pre_body,
        grid=(NPAD // RB,),
        in_specs=[
            pl.BlockSpec((RB, F), lambda i: (i, 0)),
            pl.BlockSpec((F, F), lambda i: (0, 0)),
            pl.BlockSpec((1, F), lambda i: (0, 0)),
            pl.BlockSpec((1, F), lambda i: (0, 0)),
        ],
        out_specs=[
            pl.BlockSpec((RB, F), lambda i: (i, 0)),
            pl.BlockSpec((RB, 1), lambda i: (i, 0)),
            pl.BlockSpec((RB, 1), lambda i: (i, 0)),
        ],
        out_shape=[
            jax.ShapeDtypeStruct((NPAD, F), jnp.float32),
            jax.ShapeDtypeStruct((NPAD, 1), jnp.float32),
            jax.ShapeDtypeStruct((NPAD, 1), jnp.float32),
        ],
    )(h, W, asrc, adst)


def _wts_body(as_hbm, ad_hbm, src_hbm, dst_hbm, ex_out, den_out,
              asb, adb, srcb, dstb, exs, denb):
    cid = lax.axis_index("c")
    sid = lax.axis_index("s")
    wid = sid * NC + cid

    def zero_den(i, carry):
        denb[pl.ds(i * LANES, LANES)] = jnp.zeros((LANES,), jnp.float32)
        return carry
    lax.fori_loop(0, NPAD // LANES, zero_den, 0)

    pltpu.sync_copy(as_hbm, asb)
    pltpu.sync_copy(ad_hbm, adb)

    ncw = jnp.where(cid == 0, CH0, CH1)
    ebase = jnp.where(cid == 0, sid * CH0, NS * CH0 + sid * CH1) * C

    def step(s, carry):
        base = ebase + s * C1
        pltpu.sync_copy(src_hbm.at[pl.ds(base, C1)], srcb)
        pltpu.sync_copy(dst_hbm.at[pl.ds(base, C1)], dstb)
        for ii in range(C1 // LANES):
            sv = srcb[pl.ds(ii * LANES, LANES)]
            dv = dstb[pl.ds(ii * LANES, LANES)]
            a = plsc.load_gather(asb, [sv]) + plsc.load_gather(adb, [dv])
            a = jnp.where(a >= 0, a, a * NEG_SLOPE)
            ex = jnp.exp(a)
            exs[pl.ds(ii * LANES, LANES)] = ex
            plsc.addupdate_scatter(denb, [dv], ex)
        pltpu.sync_copy(exs, ex_out.at[pl.ds(base, C1)])
        return carry
    lax.fori_loop(0, ncw // (C1 // C), step, 0)

    pltpu.sync_copy(denb, den_out.at[wid])


@functools.cache
def _wts_kernel():
  return pl.kernel(
    _wts_body,
    out_type=[
        jax.ShapeDtypeStruct((EPAD,), jnp.float32),
        jax.ShapeDtypeStruct((NW, NPAD), jnp.float32),
    ],
    mesh=plsc.VectorSubcoreMesh(core_axis_name="c", subcore_axis_name="s",
                                num_cores=NC, num_subcores=NS),
    compiler_params=pltpu.CompilerParams(needs_layout_passes=False),
    scratch_types=[
        pltpu.VMEM((NPAD,), jnp.float32),
        pltpu.VMEM((NPAD,), jnp.float32),
        pltpu.VMEM((C1,), jnp.int32),
        pltpu.VMEM((C1,), jnp.int32),
        pltpu.VMEM((C1,), jnp.float32),
        pltpu.VMEM((NPAD,), jnp.float32),
    ],
  )


def _sct_body(xw_hbm, ex_hbm, src_hbm, dst_hbm, acc_out,
              srcb, dstb, exgb, rows0, rows1, acc, semg0, semg1, sems0, sems1):
    cid = lax.axis_index("c")
    sid = lax.axis_index("s")
    wid = sid * NC + cid

    def zero_rows(r, carry):
        for j in range(F // LANES):
            rows0[r, pl.ds(j * LANES, LANES)] = jnp.zeros((LANES,), jnp.float32)
        return carry
    lax.fori_loop(0, C, zero_rows, 0)
    for k in range(STRIPE // C):
        pltpu.sync_copy(rows0, acc.at[pl.ds(sid * STRIPE + k * C, C)])
    plsc.subcore_barrier()

    ncw = jnp.where(cid == 0, CH0, CH1)
    cbase = jnp.where(cid == 0, sid * CH0, NS * CH0 + sid * CH1)

    def load_group(g, slot):
        pltpu.sync_copy(src_hbm.at[pl.ds(cbase + g * G, G)], srcb.at[pl.ds(slot * G, G)])
        pltpu.sync_copy(dst_hbm.at[pl.ds(cbase + g * G, G)], dstb.at[pl.ds(slot * G, G)])
        pltpu.sync_copy(ex_hbm.at[pl.ds(cbase + g * G, G)], exgb.at[pl.ds(slot * G, G)])

    load_group(0, 0)
    pltpu.async_copy(xw_hbm.at[srcb.at[0]], rows0, semg0)
    pltpu.async_copy(xw_hbm.at[srcb.at[1]], rows1, semg1)

    def group(g, carry):
        slot = lax.rem(g, 2)

        @pl.when(g + 1 < ncw // G)
        def _():
            load_group(g + 1, lax.rem(g + 1, 2))

        for j in range(G):
            buf, semg, sems = ((rows0, semg0, sems0) if j % 2 == 0
                               else (rows1, semg1, sems1))
            row = slot * G + j
            kk = g * G + j
            pltpu.make_async_copy(xw_hbm.at[srcb.at[row]], buf, semg).wait()

            def scale_row(i, c2):
                r0 = i * 2
                r1 = i * 2 + 1
                s0 = plsc.load_gather(exgb.at[row], [jnp.full((LANES,), r0, jnp.int32)])
                s1 = plsc.load_gather(exgb.at[row], [jnp.full((LANES,), r1, jnp.int32)])
                for jj in range(F // LANES):
                    buf[r0, pl.ds(jj * LANES, LANES)] = buf[r0, pl.ds(jj * LANES, LANES)] * s0
                    buf[r1, pl.ds(jj * LANES, LANES)] = buf[r1, pl.ds(jj * LANES, LANES)] * s1
                return c2
            lax.fori_loop(0, C // 2, scale_row, 0)

            pltpu.async_copy(buf, acc.at[dstb.at[row]], sems, add=True)
            pltpu.make_async_copy(buf, acc.at[dstb.at[row]], sems).wait()

            @pl.when(kk + 2 < ncw)
            def _():
                nrow = lax.rem(kk + 2, 2 * G)
                pltpu.async_copy(xw_hbm.at[srcb.at[nrow]], buf, semg)
        return carry

    lax.fori_loop(0, ncw // G, group, 0)
    plsc.subcore_barrier()

    for k in range(STRIPE // C):
        r0 = sid * STRIPE + k * C
        pltpu.sync_copy(acc.at[pl.ds(r0, C)], rows0)
        pltpu.sync_copy(rows0, acc_out.at[cid, pl.ds(r0, C)])


@functools.cache
def _sct_kernel():
  return pl.kernel(
    _sct_body,
    out_type=jax.ShapeDtypeStruct((NC, NPAD, F), jnp.float32),
    mesh=plsc.VectorSubcoreMesh(core_axis_name="c", subcore_axis_name="s",
                                num_cores=NC, num_subcores=NS),
    compiler_params=pltpu.CompilerParams(needs_layout_passes=False),
    scratch_types=[
        pltpu.VMEM((2 * G, C), jnp.int32),
        pltpu.VMEM((2 * G, C), jnp.int32),
        pltpu.VMEM((2 * G, C), jnp.float32),
        pltpu.VMEM((C, F), jnp.float32),
        pltpu.VMEM((C, F), jnp.float32),
        pltpu.VMEM_SHARED((NPAD, F), jnp.float32),
        pltpu.SemaphoreType.DMA,
        pltpu.SemaphoreType.DMA,
        pltpu.SemaphoreType.DMA,
        pltpu.SemaphoreType.DMA,
    ],
  )


def _edge(xw, a_s, a_d, srcp, dstp):
    ex, den2 = _wts_kernel()(a_s.reshape(NPAD), a_d.reshape(NPAD), srcp, dstp)
    acc2 = _sct_kernel()(xw, ex.reshape(EPAD // C, C),
                         srcp.reshape(EPAD // C, C), dstp.reshape(EPAD // C, C))
    return acc2, den2


def _post_body(acc_ref, den_ref, xw_ref, as_ref, ad_ref, b_ref, h_ref):
    a = as_ref[...] + ad_ref[...]
    a = jnp.where(a >= 0, a, a * NEG_SLOPE)
    exs = jnp.exp(a)
    num = acc_ref[0] + acc_ref[1] + exs * xw_ref[...]
    den = jnp.sum(den_ref[...], axis=1, keepdims=True) + exs + 1e-16
    o = num / den + b_ref[...]
    h_ref[...] = jnp.where(o > 0, o, jnp.exp(o) - 1.0)


def _post(acc2, denc, xw, a_s, a_d, b):
    return pl.pallas_call(
        _post_body,
        grid=(NPAD // RB,),
        in_specs=[
            pl.BlockSpec((NC, RB, F), lambda i: (0, i, 0)),
            pl.BlockSpec((RB, NW), lambda i: (i, 0)),
            pl.BlockSpec((RB, F), lambda i: (i, 0)),
            pl.BlockSpec((RB, 1), lambda i: (i, 0)),
            pl.BlockSpec((RB, 1), lambda i: (i, 0)),
            pl.BlockSpec((1, F), lambda i: (0, 0)),
        ],
        out_specs=pl.BlockSpec((RB, F), lambda i: (i, 0)),
        out_shape=jax.ShapeDtypeStruct((NPAD, F), jnp.float32),
    )(acc2, denc, xw, a_s, a_d, b)


def kernel(x, edge_index, W1, a_src1, a_dst1, b1, W2, a_src2, a_dst2, b2,
           W3, a_src3, a_dst3, b3):
    pad = jnp.full((EPAD - E,), PADNODE, jnp.int32)
    srcp = jnp.concatenate([edge_index[0].astype(jnp.int32), pad])
    dstp = jnp.concatenate([edge_index[1].astype(jnp.int32), pad])
    h = jnp.pad(x, ((0, NPAD - N), (0, 0)))

    for W, asrc, adst, b in ((W1, a_src1, a_dst1, b1),
                             (W2, a_src2, a_dst2, b2),
                             (W3, a_src3, a_dst3, b3)):
        xw, a_s, a_d = _pre(h, W, asrc, adst)
        acc2, den2 = _edge(xw, a_s, a_d, srcp, dstp)
        h = _post(acc2, den2.T, xw, a_s, a_d, b.reshape(1, F))
    return h[:N]


# speedup vs baseline: 1.1323x; 1.1323x over previous
import functools

import jax
import jax.numpy as jnp
from jax import lax
from jax.experimental import pallas as pl
from jax.experimental.pallas import tpu as pltpu
from jax.experimental.pallas import tpu_sc as plsc

N = 10000
E = 320000
F = 128
NEG_SLOPE = 0.2

NC = 2
NS = 16
NW = NC * NS
LANES = 16

NPAD = 10240
EPAD = 327680
PADNODE = 10200
C = 128
EPW = EPAD // NW
NCHUNK = EPW // C
G = 8
NG = NCHUNK // G
C1 = 1024
CH0 = 104
CH1 = 160 - CH0
STRIPE = NPAD // NS
RB = 1280


def _pre_body(h_ref, w_ref, asr_ref, adr_ref, xw_ref, as_ref, ad_ref):
    xw = jnp.dot(h_ref[...], w_ref[...], preferred_element_type=jnp.float32)
    xw_ref[...] = xw
    as_ref[...] = jnp.sum(xw * asr_ref[...], axis=1, keepdims=True)
    ad_ref[...] = jnp.sum(xw * adr_ref[...], axis=1, keepdims=True)


def _pre(h, W, asrc, adst):
    return pl.pallas_call(
        _pre_body,
        grid=(NPAD // RB,),
        in_specs=[
            pl.BlockSpec((RB, F), lambda i: (i, 0)),
            pl.BlockSpec((F, F), lambda i: (0, 0)),
            pl.BlockSpec((1, F), lambda i: (0, 0)),
            pl.BlockSpec((1, F), lambda i: (0, 0)),
        ],
        out_specs=[
            pl.BlockSpec((RB, F), lambda i: (i, 0)),
            pl.BlockSpec((RB, 1), lambda i: (i, 0)),
            pl.BlockSpec((RB, 1), lambda i: (i, 0)),
        ],
        out_shape=[
            jax.ShapeDtypeStruct((NPAD, F), jnp.float32),
            jax.ShapeDtypeStruct((NPAD, 1), jnp.float32),
            jax.ShapeDtypeStruct((NPAD, 1), jnp.float32),
        ],
    )(h, W, asrc, adst)


def _wts_body(as_hbm, ad_hbm, src_hbm, dst_hbm, ex_out, den_out,
              asb, adb, srcb, dstb, exs, denb):
    cid = lax.axis_index("c")
    sid = lax.axis_index("s")
    wid = sid * NC + cid

    def zero_den(i, carry):
        denb[pl.ds(i * LANES, LANES)] = jnp.zeros((LANES,), jnp.float32)
        return carry
    lax.fori_loop(0, NPAD // LANES, zero_den, 0)

    pltpu.sync_copy(as_hbm, asb)
    pltpu.sync_copy(ad_hbm, adb)

    ncw = jnp.where(cid == 0, CH0, CH1)
    ebase = jnp.where(cid == 0, sid * CH0, NS * CH0 + sid * CH1) * C

    def step(s, carry):
        base = ebase + s * C1
        pltpu.sync_copy(src_hbm.at[pl.ds(base, C1)], srcb)
        pltpu.sync_copy(dst_hbm.at[pl.ds(base, C1)], dstb)
        for ii in range(C1 // LANES):
            sv = srcb[pl.ds(ii * LANES, LANES)]
            dv = dstb[pl.ds(ii * LANES, LANES)]
            a = plsc.load_gather(asb, [sv]) + plsc.load_gather(adb, [dv])
            a = jnp.where(a >= 0, a, a * NEG_SLOPE)
            ex = jnp.exp(a)
            exs[pl.ds(ii * LANES, LANES)] = ex
            plsc.addupdate_scatter(denb, [dv], ex)
        pltpu.sync_copy(exs, ex_out.at[pl.ds(base, C1)])
        return carry
    lax.fori_loop(0, ncw // (C1 // C), step, 0)

    pltpu.sync_copy(denb, den_out.at[wid])


@functools.cache
def _wts_kernel():
  return pl.kernel(
    _wts_body,
    out_type=[
        jax.ShapeDtypeStruct((EPAD,), jnp.float32),
        jax.ShapeDtypeStruct((NW, NPAD), jnp.float32),
    ],
    mesh=plsc.VectorSubcoreMesh(core_axis_name="c", subcore_axis_name="s",
                                num_cores=NC, num_subcores=NS),
    compiler_params=pltpu.CompilerParams(needs_layout_passes=False),
    scratch_types=[
        pltpu.VMEM((NPAD,), jnp.float32),
        pltpu.VMEM((NPAD,), jnp.float32),
        pltpu.VMEM((C1,), jnp.int32),
        pltpu.VMEM((C1,), jnp.int32),
        pltpu.VMEM((C1,), jnp.float32),
        pltpu.VMEM((NPAD,), jnp.float32),
    ],
  )


def _sct_body(xw_hbm, ex_hbm, src_hbm, dst_hbm, acc_out,
              srcb, dstb, exgb, rows0, rows1, acc, semg0, semg1, sems0, sems1):
    cid = lax.axis_index("c")
    sid = lax.axis_index("s")
    wid = sid * NC + cid

    def zero_rows(r, carry):
        for j in range(F // LANES):
            rows0[r, pl.ds(j * LANES, LANES)] = jnp.zeros((LANES,), jnp.float32)
        return carry
    lax.fori_loop(0, C, zero_rows, 0)
    for k in range(STRIPE // C):
        pltpu.sync_copy(rows0, acc.at[pl.ds(sid * STRIPE + k * C, C)])
    plsc.subcore_barrier()

    ncw = jnp.where(cid == 0, CH0, CH1)
    cbase = jnp.where(cid == 0, sid * CH0, NS * CH0 + sid * CH1)

    def load_group(g, slot):
        pltpu.sync_copy(src_hbm.at[pl.ds(cbase + g * G, G)], srcb.at[pl.ds(slot * G, G)])
        pltpu.sync_copy(dst_hbm.at[pl.ds(cbase + g * G, G)], dstb.at[pl.ds(slot * G, G)])
        pltpu.sync_copy(ex_hbm.at[pl.ds(cbase + g * G, G)], exgb.at[pl.ds(slot * G, G)])

    load_group(0, 0)
    pltpu.async_copy(xw_hbm.at[srcb.at[0]], rows0, semg0)
    pltpu.async_copy(xw_hbm.at[srcb.at[1]], rows1, semg1)

    def group(g, carry):
        slot = lax.rem(g, 2)

        @pl.when(g + 1 < ncw // G)
        def _():
            load_group(g + 1, lax.rem(g + 1, 2))

        for j in range(G):
            buf, semg, sems = ((rows0, semg0, sems0) if j % 2 == 0
                               else (rows1, semg1, sems1))
            row = slot * G + j
            kk = g * G + j
            pltpu.make_async_copy(xw_hbm.at[srcb.at[row]], buf, semg).wait()

            def scale_row(i, c2):
                r0 = i * 2
                r1 = i * 2 + 1
                s0 = plsc.load_gather(exgb.at[row], [jnp.full((LANES,), r0, jnp.int32)])
                s1 = plsc.load_gather(exgb.at[row], [jnp.full((LANES,), r1, jnp.int32)])
                for jj in range(F // LANES):
                    buf[r0, pl.ds(jj * LANES, LANES)] = buf[r0, pl.ds(jj * LANES, LANES)] * s0
                    buf[r1, pl.ds(jj * LANES, LANES)] = buf[r1, pl.ds(jj * LANES, LANES)] * s1
                return c2
            lax.fori_loop(0, C // 2, scale_row, 0)

            pltpu.async_copy(buf, acc.at[dstb.at[row]], sems, add=True)
            pltpu.make_async_copy(buf, acc.at[dstb.at[row]], sems).wait()

            @pl.when(kk + 2 < ncw)
            def _():
                nrow = lax.rem(kk + 2, 2 * G)
                pltpu.async_copy(xw_hbm.at[srcb.at[nrow]], buf, semg)
        return carry

    lax.fori_loop(0, ncw // G, group, 0)
    plsc.subcore_barrier()

    for k in range(STRIPE // C):
        r0 = sid * STRIPE + k * C
        pltpu.sync_copy(acc.at[pl.ds(r0, C)], rows0)
        pltpu.sync_copy(rows0, acc_out.at[cid, pl.ds(r0, C)])


@functools.cache
def _sct_kernel():
  return pl.kernel(
    _sct_body,
    out_type=jax.ShapeDtypeStruct((NC, NPAD, F), jnp.float32),
    mesh=plsc.VectorSubcoreMesh(core_axis_name="c", subcore_axis_name="s",
                                num_cores=NC, num_subcores=NS),
    compiler_params=pltpu.CompilerParams(needs_layout_passes=False),
    scratch_types=[
        pltpu.VMEM((2 * G, C), jnp.int32),
        pltpu.VMEM((2 * G, C), jnp.int32),
        pltpu.VMEM((2 * G, C), jnp.float32),
        pltpu.VMEM((C, F), jnp.float32),
        pltpu.VMEM((C, F), jnp.float32),
        pltpu.VMEM_SHARED((NPAD, F), jnp.float32),
        pltpu.SemaphoreType.DMA,
        pltpu.SemaphoreType.DMA,
        pltpu.SemaphoreType.DMA,
        pltpu.SemaphoreType.DMA,
    ],
  )


def _edge(xw, a_s, a_d, srcp, dstp):
    ex, den2 = _wts_kernel()(a_s.reshape(NPAD), a_d.reshape(NPAD), srcp, dstp)
    acc2 = _sct_kernel()(xw, ex.reshape(EPAD // C, C),
                         srcp.reshape(EPAD // C, C), dstp.reshape(EPAD // C, C))
    return acc2, den2


def _post_body(acc_ref, den_ref, xw_ref, as_ref, ad_ref, b_ref, h_ref):
    a = as_ref[...] + ad_ref[...]
    a = jnp.where(a >= 0, a, a * NEG_SLOPE)
    exs = jnp.exp(a)
    num = acc_ref[0] + acc_ref[1] + exs * xw_ref[...]
    den = jnp.sum(den_ref[...], axis=1, keepdims=True) + exs + 1e-16
    o = num / den + b_ref[...]
    h_ref[...] = jnp.where(o > 0, o, jnp.exp(o) - 1.0)


def _post(acc2, denc, xw, a_s, a_d, b):
    return pl.pallas_call(
        _post_body,
        grid=(NPAD // RB,),
        in_specs=[
            pl.BlockSpec((NC, RB, F), lambda i: (0, i, 0)),
            pl.BlockSpec((RB, NW), lambda i: (i, 0)),
            pl.BlockSpec((RB, F), lambda i: (i, 0)),
            pl.BlockSpec((RB, 1), lambda i: (i, 0)),
            pl.BlockSpec((RB, 1), lambda i: (i, 0)),
            pl.BlockSpec((1, F), lambda i: (0, 0)),
        ],
        out_specs=pl.BlockSpec((RB, F), lambda i: (i, 0)),
        out_shape=jax.ShapeDtypeStruct((NPAD, F), jnp.float32),
    )(acc2, denc, xw, a_s, a_d, b)


def kernel(x, edge_index, W1, a_src1, a_dst1, b1, W2, a_src2, a_dst2, b2,
           W3, a_src3, a_dst3, b3):
    pad = jnp.full((EPAD - E,), PADNODE, jnp.int32)
    srcp = jnp.concatenate([edge_index[0].astype(jnp.int32), pad])
    dstp = jnp.concatenate([edge_index[1].astype(jnp.int32), pad])
    h = jnp.pad(x, ((0, NPAD - N), (0, 0)))

    for W, asrc, adst, b in ((W1, a_src1, a_dst1, b1),
                             (W2, a_src2, a_dst2, b2),
                             (W3, a_src3, a_dst3, b3)):
        xw, a_s, a_d = _pre(h, W, asrc, adst)
        acc2, den2 = _edge(xw, a_s, a_d, srcp, dstp)
        h = _post(acc2, den2.T, xw, a_s, a_d, b.reshape(1, F))
    return h[:N]
